# baseline (speedup 1.0000x reference)
import functools

import jax
import jax.numpy as jnp
from jax import lax
from jax.experimental import pallas as pl
from jax.experimental.pallas import tpu as pltpu
from jax.experimental.pallas import tpu_sc as plsc

_POINT = 11
_PAD_P = 16
_ACT = 4
_GROUPS = ((0, 4), (4, 9), (9, 10), (10, 11))

_SC_INFO = plsc.get_sparse_core_info()
_NC = _SC_INFO.num_cores
_NW = _NC * _SC_INFO.num_subcores
_LANES = _SC_INFO.num_lanes


def _tc_body(x_ref, av_ref, out_ref):
    x = x_ref[...]
    av = av_ref[...]
    avn2 = jnp.sum(av * av, axis=0, keepdims=True)
    avs = av / jnp.maximum(jnp.sqrt(avn2), 1e-8)
    numT = lax.dot_general(avs, x, (((0,), (1,)), ((), ())),
                           preferred_element_type=jnp.float32)
    ones = jnp.ones((1, x_ref.shape[1]), jnp.float32)
    ssT = lax.dot_general(ones, x * x, (((1,), (1,)), ((), ())),
                          preferred_element_type=jnp.float32)
    out_ref[...] = (numT / jnp.maximum(jnp.sqrt(ssT), 1e-8))[None]


def _sc_body(sims_hbm, wb_hbm, out_hbm, sims_v, wb_v, out_v):
    cols = sims_v.shape[1]
    wid = lax.axis_index("s") * _NC + lax.axis_index("c")
    pltpu.sync_copy(sims_hbm.at[wid], sims_v)
    pltpu.sync_copy(wb_hbm, wb_v)
    wrows = [wb_v[j] for j in range(_ACT + 1)]
    w = [[wrows[j][k] for k in range(_ACT)] for j in range(_ACT)]
    bias = [wrows[_ACT][j] for j in range(_ACT)]
    for c in range(cols // _LANES):
        sl = pl.ds(c * _LANES, _LANES)
        v = [sims_v[r, sl] for r in range(_POINT)]
        g = []
        for (s, e) in _GROUPS:
            m = v[s]
            for r in range(s + 1, e):
                m = jnp.maximum(m, v[r])
            g.append(m)
        logits = []
        for j in range(_ACT):
            l = bias[j]
            for k in range(_ACT):
                l = l + w[j][k] * g[k]
            logits.append(l)
        m = jnp.maximum(jnp.maximum(logits[0], logits[1]),
                        jnp.maximum(logits[2], logits[3]))
        exps = [jnp.exp(l - m) for l in logits]
        tot = exps[0] + exps[1] + exps[2] + exps[3]
        for j in range(_ACT):
            out_v[j, sl] = exps[j] / tot
    pltpu.sync_copy(out_v, out_hbm.at[wid])


def kernel(embedded_words, action_vectors, W, b):
    B, D = embedded_words.shape
    cols = B // _NW
    av = jnp.pad(action_vectors[0], ((0, 0), (0, _PAD_P - _POINT)))
    wb = jnp.zeros((8, _LANES), jnp.float32).at[:_ACT, :_ACT].set(W).at[_ACT, :_ACT].set(b)

    simsT = pl.pallas_call(
        _tc_body,
        grid=(_NW,),
        in_specs=[
            pl.BlockSpec((cols, D), lambda i: (i, 0)),
            pl.BlockSpec((D, _PAD_P), lambda i: (0, 0)),
        ],
        out_specs=pl.BlockSpec((1, _PAD_P, cols), lambda i: (i, 0, 0)),
        out_shape=jax.ShapeDtypeStruct((_NW, _PAD_P, cols), jnp.float32),
    )(embedded_words, av)

    sc_fn = pl.kernel(
        _sc_body,
        out_type=jax.ShapeDtypeStruct((_NW, _ACT, cols), jnp.float32),
        mesh=plsc.VectorSubcoreMesh(core_axis_name="c", subcore_axis_name="s"),
        scratch_types=[
            pltpu.VMEM((_PAD_P, cols), jnp.float32),
            pltpu.VMEM((8, _LANES), jnp.float32),
            pltpu.VMEM((_ACT, cols), jnp.float32),
        ],
    )
    outT = sc_fn(simsT, wb)
    return jnp.transpose(outT, (0, 2, 1)).reshape(B, _ACT)


# speedup vs baseline: 1.4080x; 1.4080x over previous
import jax
import jax.numpy as jnp
from jax import lax
from jax.experimental import pallas as pl
from jax.experimental.pallas import tpu as pltpu
from jax.experimental.pallas import tpu_sc as plsc

_POINT = 11
_PAD_P = 16
_ACT = 4
_GROUPS = ((0, 4), (4, 9), (9, 10), (10, 11))

_BLK = 2048

_SC_INFO = plsc.get_sparse_core_info()
_NC = _SC_INFO.num_cores
_NW = _NC * _SC_INFO.num_subcores
_LANES = _SC_INFO.num_lanes


def _tc_body(x_ref, av_ref, out_ref):
    x = x_ref[...]
    av = av_ref[...]
    avn2 = jnp.sum(av * av, axis=0, keepdims=True)
    avs = av / jnp.maximum(jnp.sqrt(avn2), 1e-8)
    numT = lax.dot_general(avs, x, (((0,), (1,)), ((), ())),
                           preferred_element_type=jnp.float32)
    ones = jnp.ones((1, x_ref.shape[1]), jnp.float32)
    ssT = lax.dot_general(ones, x * x, (((1,), (1,)), ((), ())),
                          preferred_element_type=jnp.float32)
    out_ref[...] = numT / jnp.maximum(jnp.sqrt(ssT), 1e-8)


def _sc_body(sims_hbm, wb_hbm, out_hbm, sims_v, wb_v, out_v):
    cols = sims_v.shape[1]
    wid = lax.axis_index("s") * _NC + lax.axis_index("c")
    base = wid * cols
    pltpu.sync_copy(sims_hbm.at[:, pl.ds(base, cols)], sims_v)
    pltpu.sync_copy(wb_hbm, wb_v)
    wrows = [wb_v[j] for j in range(_ACT + 1)]
    w = [[wrows[j][k] for k in range(_ACT)] for j in range(_ACT)]
    bias = [wrows[_ACT][j] for j in range(_ACT)]
    for c in range(cols // _LANES):
        sl = pl.ds(c * _LANES, _LANES)
        v = [sims_v[r, sl] for r in range(_POINT)]
        g = []
        for (s, e) in _GROUPS:
            m = v[s]
            for r in range(s + 1, e):
                m = jnp.maximum(m, v[r])
            g.append(m)
        logits = []
        for j in range(_ACT):
            l = bias[j]
            for k in range(_ACT):
                l = l + w[j][k] * g[k]
            logits.append(l)
        m = jnp.maximum(jnp.maximum(logits[0], logits[1]),
                        jnp.maximum(logits[2], logits[3]))
        exps = [jnp.exp(l - m) for l in logits]
        inv = 1.0 / (exps[0] + exps[1] + exps[2] + exps[3])
        for j in range(_ACT):
            out_v[j, sl] = exps[j] * inv
    pltpu.sync_copy(out_v, out_hbm.at[:, pl.ds(base, cols)])


def kernel(embedded_words, action_vectors, W, b):
    B, D = embedded_words.shape
    cols = B // _NW
    av = jnp.pad(action_vectors[0], ((0, 0), (0, _PAD_P - _POINT)))
    wb = jnp.zeros((8, _LANES), jnp.float32).at[:_ACT, :_ACT].set(W).at[_ACT, :_ACT].set(b)

    simsT = pl.pallas_call(
        _tc_body,
        grid=(B // _BLK,),
        in_specs=[
            pl.BlockSpec((_BLK, D), lambda i: (i, 0)),
            pl.BlockSpec((D, _PAD_P), lambda i: (0, 0)),
        ],
        out_specs=pl.BlockSpec((_PAD_P, _BLK), lambda i: (0, i)),
        out_shape=jax.ShapeDtypeStruct((_PAD_P, B), jnp.float32),
    )(embedded_words, av)

    sc_fn = pl.kernel(
        _sc_body,
        out_type=jax.ShapeDtypeStruct((_ACT, B), jnp.float32),
        mesh=plsc.VectorSubcoreMesh(core_axis_name="c", subcore_axis_name="s"),
        scratch_types=[
            pltpu.VMEM((_PAD_P, cols), jnp.float32),
            pltpu.VMEM((8, _LANES), jnp.float32),
            pltpu.VMEM((_ACT, cols), jnp.float32),
        ],
    )
    return sc_fn(simsT, wb).T
